# baseline (speedup 1.0000x reference)
import functools

import jax
import jax.numpy as jnp
from jax import lax
from jax.experimental import pallas as pl
from jax.experimental.pallas import tpu as pltpu
from jax.experimental.pallas import tpu_sc as plsc

N = 10000
IN_DIM = 128
HID = 64
OUT = 64
NUM_WALKS = 4
WALK_LEN = 8
HEADS = 4
DH = HID // HEADS
N_EDGES = 320000
TOK = NUM_WALKS * WALK_LEN
NTOK = N * TOK
FW = 128

NPAD = 10240
NC, NS = 2, 16
NW = NC * NS
CHUNK = 80


def _t1_body(x_ref, w_ref, o_ref):
    o_ref[...] = jnp.dot(x_ref[...], w_ref[...],
                         preferred_element_type=jnp.float32)


def _project(x, w_in_pad):
    blk = 2000
    return pl.pallas_call(
        _t1_body,
        grid=(N // blk,),
        in_specs=[
            pl.BlockSpec((blk, IN_DIM), lambda i: (i, 0)),
            pl.BlockSpec((IN_DIM, FW), lambda i: (0, 0)),
        ],
        out_specs=pl.BlockSpec((blk, FW), lambda i: (i, 0)),
        out_shape=jax.ShapeDtypeStruct((N, FW), jnp.float32),
    )(x, w_in_pad)


def _gather_and_hist(xp, wflat, dst, ones_rows, zeros_rows):
    per_w = NTOK // NW
    nchunks = per_w // CHUNK
    rows_per_tile = NPAD // NS

    mesh = plsc.VectorSubcoreMesh(core_axis_name="c", subcore_axis_name="s")

    @functools.partial(
        pl.kernel, mesh=mesh,
        out_type=[
            jax.ShapeDtypeStruct((NTOK, FW), jnp.float32),
            jax.ShapeDtypeStruct((NC, NPAD, FW), jnp.float32),
        ],
        scratch_types=[
            pltpu.VMEM((CHUNK,), jnp.int32),
            pltpu.VMEM((CHUNK,), jnp.int32),
            pltpu.VMEM((CHUNK, FW), jnp.float32),
            pltpu.VMEM((CHUNK, FW), jnp.float32),
            pltpu.VMEM_SHARED((NPAD, FW), jnp.float32),
            pltpu.SemaphoreType.DMA,
        ],
    )
    def k(xp_hbm, idx_hbm, dst_hbm, ones_hbm, zeros_hbm, z_hbm, hist_hbm,
          idx_v, didx_v, rows_v, ones_v, acc_sh, sem):
        cid = lax.axis_index("c")
        sid = lax.axis_index("s")
        wid = cid * NS + sid
        rbase = pl.multiple_of(sid * rows_per_tile, 8)
        pltpu.sync_copy(zeros_hbm.at[pl.ds(0, rows_per_tile)],
                        acc_sh.at[pl.ds(rbase, rows_per_tile)])
        pltpu.sync_copy(ones_hbm, ones_v)
        plsc.subcore_barrier()

        base = pl.multiple_of(wid * per_w, 8)

        def body(j, _):
            off = pl.multiple_of(base + j * CHUNK, 8)
            pltpu.sync_copy(idx_hbm.at[pl.ds(off, CHUNK)], idx_v)
            pltpu.async_copy(xp_hbm.at[idx_v], rows_v, sem).wait()
            pltpu.sync_copy(rows_v, z_hbm.at[pl.ds(off, CHUNK)])
            pltpu.sync_copy(dst_hbm.at[pl.ds(off, CHUNK)], didx_v)
            pltpu.sync_copy(ones_v, acc_sh.at[didx_v], add=True)
            return 0

        lax.fori_loop(0, nchunks, body, 0)
        plsc.subcore_barrier()
        pltpu.sync_copy(acc_sh.at[pl.ds(rbase, rows_per_tile)],
                        hist_hbm.at[cid, pl.ds(rbase, rows_per_tile)])

    return k(xp, wflat, dst, ones_rows, zeros_rows)


def _edge_scatter(rows, src, dst, zeros_rows):
    per_w = N_EDGES // NW
    nchunks = per_w // CHUNK
    rows_per_tile = NPAD // NS

    mesh = plsc.VectorSubcoreMesh(core_axis_name="c", subcore_axis_name="s")

    @functools.partial(
        pl.kernel, mesh=mesh,
        out_type=jax.ShapeDtypeStruct((NC, NPAD, FW), jnp.float32),
        scratch_types=[
            pltpu.VMEM((CHUNK,), jnp.int32),
            pltpu.VMEM((CHUNK,), jnp.int32),
            pltpu.VMEM((CHUNK, FW), jnp.float32),
            pltpu.VMEM_SHARED((NPAD, FW), jnp.float32),
            pltpu.SemaphoreType.DMA,
        ],
    )
    def k(rows_hbm, src_hbm, dst_hbm, zeros_hbm, out_hbm,
          sidx_v, didx_v, rows_v, acc_sh, sem):
        cid = lax.axis_index("c")
        sid = lax.axis_index("s")
        wid = cid * NS + sid
        rbase = pl.multiple_of(sid * rows_per_tile, 8)
        pltpu.sync_copy(zeros_hbm.at[pl.ds(0, rows_per_tile)],
                        acc_sh.at[pl.ds(rbase, rows_per_tile)])
        plsc.subcore_barrier()

        base = pl.multiple_of(wid * per_w, 8)

        def body(j, _):
            off = pl.multiple_of(base + j * CHUNK, 8)
            pltpu.sync_copy(src_hbm.at[pl.ds(off, CHUNK)], sidx_v)
            pltpu.sync_copy(dst_hbm.at[pl.ds(off, CHUNK)], didx_v)
            pltpu.async_copy(rows_hbm.at[sidx_v], rows_v, sem).wait()
            pltpu.sync_copy(rows_v, acc_sh.at[didx_v], add=True)
            return 0

        lax.fori_loop(0, nchunks, body, 0)
        plsc.subcore_barrier()
        pltpu.sync_copy(acc_sh.at[pl.ds(rbase, rows_per_tile)],
                        out_hbm.at[cid, pl.ds(rbase, rows_per_tile)])

    return k(rows, src, dst, zeros_rows)


BN = 16
BR = BN * TOK // 2
NROW = NTOK // 2
NBLK = N // BN


def _group_roll(arr, o, pos):
    t = arr.shape[0]
    a = jnp.concatenate([arr[o:], arr[:o]], axis=0)
    r2 = t + o - WALK_LEN
    b = jnp.concatenate([arr[r2:], arr[:r2]], axis=0)
    return jnp.where(pos < WALK_LEN - o, a, b)


def _t2_body(za_ref, zb_ref, deg_ref, hist_ref, wq_ref, wk_ref, wv_ref,
             wo_ref, w1_ref, w2_ref, e2_ref, e2t_ref, wse_ref, g1_ref,
             hn1_ref, dinv_ref):
    f32 = jnp.float32
    z = jnp.concatenate([za_ref[...][:, :HID], zb_ref[...][:, :HID]], axis=1)
    q = jnp.dot(z, wq_ref[...], preferred_element_type=f32) * (1.0 / 4.0)
    kk = jnp.dot(z, wk_ref[...], preferred_element_type=f32)
    v = jnp.dot(z, wv_ref[...], preferred_element_type=f32)

    E2 = e2_ref[...]
    E2T = e2t_ref[...]
    pos = lax.broadcasted_iota(jnp.int32, (BR, FW), 0) % WALK_LEN

    ks = [kk] + [_group_roll(kk, o, pos) for o in range(1, WALK_LEN)]
    logits = [jnp.dot(q * ko, E2, preferred_element_type=f32) for ko in ks]
    lcat = jnp.concatenate(logits, axis=1)
    m = jnp.maximum(lcat[:, :32], lcat[:, 32:])
    m = jnp.maximum(m[:, :16], m[:, 16:])
    m = jnp.maximum(m[:, :8], m[:, 8:])
    mt = jnp.concatenate([m] * WALK_LEN, axis=1)
    e = jnp.exp(lcat - mt)
    d = e[:, :32] + e[:, 32:]
    d = d[:, :16] + d[:, 16:]
    inv_den = 1.0 / (d[:, :8] + d[:, 8:])
    it = jnp.concatenate([inv_den] * WALK_LEN, axis=1)
    a = e * it
    o_acc = None
    for o in range(WALK_LEN):
        a_full = jnp.dot(a[:, 8 * o:8 * o + 8], E2T,
                         preferred_element_type=f32)
        vo = v if o == 0 else _group_roll(v, o, pos)
        contrib = a_full * vo
        o_acc = contrib if o_acc is None else o_acc + contrib
    z = z + jnp.dot(o_acc, wo_ref[...], preferred_element_type=f32)
    h1 = jnp.maximum(jnp.dot(z, w1_ref[...], preferred_element_type=f32), 0.0)
    z = z + jnp.dot(h1, w2_ref[...], preferred_element_type=f32)

    ri = lax.broadcasted_iota(jnp.int32, (BN, BR), 0)
    ci = lax.broadcasted_iota(jnp.int32, (BN, BR), 1) // (TOK // 2)
    P = jnp.where(ri == ci, 1.0 / TOK, 0.0).astype(f32)
    p128 = jnp.dot(P, z, preferred_element_type=f32)
    pooled = p128[:, :HID] + p128[:, HID:]

    deg = deg_ref[...]
    gf = 1.0 + jnp.log1p(jnp.maximum(deg, 0.0)) * wse_ref[...]
    gt = jnp.maximum(pooled * gf, 0.0)

    h = jnp.dot(gt, g1_ref[...], preferred_element_type=f32)
    degc = hist_ref[..., 0:1] + hist_ref[..., 1:2] + 1.0
    dinv = lax.rsqrt(degc)
    hn1_ref[...] = h * dinv
    dinv_ref[...] = dinv


def _bd(w):
    a, b = w.shape
    z = jnp.zeros((a, b), w.dtype)
    return jnp.concatenate([
        jnp.concatenate([w, z], axis=1),
        jnp.concatenate([z, w], axis=1),
    ], axis=0)


def _transformer(z, deg2, hist2, wq, wk, wv, wo, w1, w2, wse2, gcn1_w):
    f32 = jnp.float32
    bd_wq, bd_wk, bd_wv, bd_wo = _bd(wq), _bd(wk), _bd(wv), _bd(wo)
    bd_w1, bd_w2 = _bd(w1), _bd(w2)
    li = jnp.arange(FW)
    hj = jnp.arange(2 * HEADS)
    e2 = (li[:, None] // DH == hj[None, :]).astype(f32)
    e2t = e2.T
    wcon = pl.BlockSpec((FW, FW), lambda i: (0, 0))
    return pl.pallas_call(
        _t2_body,
        grid=(NBLK,),
        in_specs=[
            pl.BlockSpec((BR, FW), lambda i: (i, 0)),
            pl.BlockSpec((BR, FW), lambda i: (i + NBLK, 0)),
            pl.BlockSpec((BN, 1), lambda i: (i, 0)),
            pl.BlockSpec((BN, 2), lambda i: (i, 0)),
            wcon, wcon, wcon, wcon,
            pl.BlockSpec((FW, 2 * FW), lambda i: (0, 0)),
            pl.BlockSpec((2 * FW, FW), lambda i: (0, 0)),
            pl.BlockSpec((FW, 2 * HEADS), lambda i: (0, 0)),
            pl.BlockSpec((2 * HEADS, FW), lambda i: (0, 0)),
            pl.BlockSpec((1, HID), lambda i: (0, 0)),
            pl.BlockSpec((HID, 2 * OUT), lambda i: (0, 0)),
        ],
        out_specs=[
            pl.BlockSpec((BN, 2 * OUT), lambda i: (i, 0)),
            pl.BlockSpec((BN, 1), lambda i: (i, 0)),
        ],
        out_shape=[
            jax.ShapeDtypeStruct((N, 2 * OUT), jnp.float32),
            jax.ShapeDtypeStruct((N, 1), jnp.float32),
        ],
    )(z, z, deg2, hist2, bd_wq, bd_wk, bd_wv, bd_wo, bd_w1, bd_w2,
      e2, e2t, wse2, gcn1_w)


def _t3_body(p0_ref, p1_ref, hn1_ref, dinv_ref, b1_ref, g2_ref, hn2_ref):
    dinv = dinv_ref[...]
    s = p0_ref[...] + p1_ref[...] + hn1_ref[...]
    h1 = jnp.maximum(dinv * s + b1_ref[...], 0.0)
    hn2_ref[...] = dinv * jnp.dot(h1, g2_ref[...],
                                  preferred_element_type=jnp.float32)


def _gcn_mid(p0, p1, hn1, dinv, b1_2, gcn2_w_pad):
    blk = 2000
    return pl.pallas_call(
        _t3_body,
        grid=(N // blk,),
        in_specs=[
            pl.BlockSpec((blk, FW), lambda i: (i, 0)),
            pl.BlockSpec((blk, FW), lambda i: (i, 0)),
            pl.BlockSpec((blk, 2 * OUT), lambda i: (i, 0)),
            pl.BlockSpec((blk, 1), lambda i: (i, 0)),
            pl.BlockSpec((1, 2 * OUT), lambda i: (0, 0)),
            pl.BlockSpec((2 * OUT, FW), lambda i: (0, 0)),
        ],
        out_specs=pl.BlockSpec((blk, FW), lambda i: (i, 0)),
        out_shape=jax.ShapeDtypeStruct((N, FW), jnp.float32),
    )(p0, p1, hn1, dinv, b1_2, gcn2_w_pad)


def _t4_body(p0_ref, p1_ref, hn2_ref, dinv_ref, b2_ref, o_ref):
    s = p0_ref[...] + p1_ref[...] + hn2_ref[...]
    s = dinv_ref[...] * s[:, :OUT] + b2_ref[...]
    m = jnp.max(s, axis=1, keepdims=True)
    e = jnp.exp(s - m)
    o_ref[...] = e / jnp.sum(e, axis=1, keepdims=True)


def _finalize(p0, p1, hn2, dinv, b2_2):
    blk = 2000
    return pl.pallas_call(
        _t4_body,
        grid=(N // blk,),
        in_specs=[
            pl.BlockSpec((blk, FW), lambda i: (i, 0)),
            pl.BlockSpec((blk, FW), lambda i: (i, 0)),
            pl.BlockSpec((blk, FW), lambda i: (i, 0)),
            pl.BlockSpec((blk, 1), lambda i: (i, 0)),
            pl.BlockSpec((1, OUT), lambda i: (0, 0)),
        ],
        out_specs=pl.BlockSpec((blk, OUT), lambda i: (i, 0)),
        out_shape=jax.ShapeDtypeStruct((N, OUT), jnp.float32),
    )(p0, p1, hn2, dinv, b2_2)


def kernel(x, deg, edge_index, walks, w_in, wq, wk, wv, wo, w1, w2, w_se,
           gcn1_w, gcn1_b, gcn2_w, gcn2_b):
    f32 = jnp.float32
    wflat = jnp.concatenate([
        walks[:, :2, :].reshape(-1), walks[:, 2:, :].reshape(-1)
    ]).astype(jnp.int32)
    src = edge_index[0].astype(jnp.int32)
    dst = edge_index[1].astype(jnp.int32)

    ones_rows = jnp.ones((CHUNK, FW), f32)
    zeros_rows = jnp.zeros((NPAD // NS, FW), f32)

    w_in_pad = jnp.pad(w_in, ((0, 0), (0, FW - HID)))
    xp = _project(x, w_in_pad)
    z, hist = _gather_and_hist(xp, wflat, dst, ones_rows, zeros_rows)
    hist2 = hist[:, :N, 0].T

    hn1, dinv = _transformer(
        z, deg.reshape(N, 1), hist2, wq[0], wk[0], wv[0], wo[0],
        w1[0], w2[0], w_se.reshape(1, HID), gcn1_w)

    s1 = _edge_scatter(hn1, src, dst, zeros_rows)
    gcn2_w_pad = jnp.pad(gcn2_w, ((0, 0), (0, FW - OUT)))
    hn2 = _gcn_mid(s1[0, :N], s1[1, :N], hn1, dinv,
                   gcn1_b.reshape(1, 2 * OUT), gcn2_w_pad)

    s2 = _edge_scatter(hn2, src, dst, zeros_rows)
    return _finalize(s2[0, :N], s2[1, :N], hn2, dinv,
                     gcn2_b.reshape(1, OUT))


# speedup vs baseline: 1.5431x; 1.5431x over previous
import functools

import jax
import jax.numpy as jnp
from jax import lax
from jax.experimental import pallas as pl
from jax.experimental.pallas import tpu as pltpu
from jax.experimental.pallas import tpu_sc as plsc

N = 10000
IN_DIM = 128
HID = 64
OUT = 64
NUM_WALKS = 4
WALK_LEN = 8
HEADS = 4
DH = HID // HEADS
N_EDGES = 320000
TOK = NUM_WALKS * WALK_LEN
NTOK = N * TOK
FW = 128

NPAD = 10240
NC, NS = 2, 16
NW = NC * NS
CHUNK = 80


def _t1_body(x_ref, w_ref, o_ref):
    o_ref[...] = jnp.dot(x_ref[...], w_ref[...],
                         preferred_element_type=jnp.float32)


def _project(x, w_in_pad):
    blk = 2000
    return pl.pallas_call(
        _t1_body,
        grid=(N // blk,),
        in_specs=[
            pl.BlockSpec((blk, IN_DIM), lambda i: (i, 0)),
            pl.BlockSpec((IN_DIM, FW), lambda i: (0, 0)),
        ],
        out_specs=pl.BlockSpec((blk, FW), lambda i: (i, 0)),
        out_shape=jax.ShapeDtypeStruct((N, FW), jnp.float32),
    )(x, w_in_pad)


def _gather_and_hist(xp, wflat, dst, ones_rows, zeros_rows):
    per_w = NTOK // NW
    nchunks = per_w // CHUNK
    rows_per_tile = NPAD // NS

    mesh = plsc.VectorSubcoreMesh(core_axis_name="c", subcore_axis_name="s")

    @functools.partial(
        pl.kernel, mesh=mesh,
        out_type=[
            jax.ShapeDtypeStruct((NTOK, FW), jnp.float32),
            jax.ShapeDtypeStruct((NC, NPAD, FW), jnp.float32),
        ],
        scratch_types=[
            pltpu.VMEM((CHUNK,), jnp.int32),
            pltpu.VMEM((CHUNK,), jnp.int32),
            pltpu.VMEM((CHUNK, FW), jnp.float32),
            pltpu.VMEM((CHUNK, FW), jnp.float32),
            pltpu.VMEM_SHARED((NPAD, FW), jnp.float32),
            pltpu.SemaphoreType.DMA,
        ],
    )
    def k(xp_hbm, idx_hbm, dst_hbm, ones_hbm, zeros_hbm, z_hbm, hist_hbm,
          idx_v, didx_v, rows_v, ones_v, acc_sh, sem):
        cid = lax.axis_index("c")
        sid = lax.axis_index("s")
        wid = cid * NS + sid
        rbase = pl.multiple_of(sid * rows_per_tile, 8)
        pltpu.sync_copy(zeros_hbm.at[pl.ds(0, rows_per_tile)],
                        acc_sh.at[pl.ds(rbase, rows_per_tile)])
        pltpu.sync_copy(ones_hbm, ones_v)
        plsc.subcore_barrier()

        base = pl.multiple_of(wid * per_w, 8)

        def body(j, _):
            off = pl.multiple_of(base + j * CHUNK, 8)
            pltpu.sync_copy(idx_hbm.at[pl.ds(off, CHUNK)], idx_v)
            pltpu.async_copy(xp_hbm.at[idx_v], rows_v, sem).wait()
            pltpu.sync_copy(rows_v, z_hbm.at[pl.ds(off, CHUNK)])
            pltpu.sync_copy(dst_hbm.at[pl.ds(off, CHUNK)], didx_v)
            pltpu.sync_copy(ones_v, acc_sh.at[didx_v], add=True)
            return 0

        lax.fori_loop(0, nchunks, body, 0)
        plsc.subcore_barrier()
        pltpu.sync_copy(acc_sh.at[pl.ds(rbase, rows_per_tile)],
                        hist_hbm.at[cid, pl.ds(rbase, rows_per_tile)])

    return k(xp, wflat, dst, ones_rows, zeros_rows)


def _edge_scatter(rows, src, dst, zeros_rows):
    per_w = N_EDGES // NW
    nchunks = per_w // CHUNK
    rows_per_tile = NPAD // NS

    mesh = plsc.VectorSubcoreMesh(core_axis_name="c", subcore_axis_name="s")

    @functools.partial(
        pl.kernel, mesh=mesh,
        out_type=jax.ShapeDtypeStruct((NC, NPAD, FW), jnp.float32),
        scratch_types=[
            pltpu.VMEM((CHUNK,), jnp.int32),
            pltpu.VMEM((CHUNK,), jnp.int32),
            pltpu.VMEM((CHUNK, FW), jnp.float32),
            pltpu.VMEM_SHARED((NPAD, FW), jnp.float32),
            pltpu.SemaphoreType.DMA,
        ],
    )
    def k(rows_hbm, src_hbm, dst_hbm, zeros_hbm, out_hbm,
          sidx_v, didx_v, rows_v, acc_sh, sem):
        cid = lax.axis_index("c")
        sid = lax.axis_index("s")
        wid = cid * NS + sid
        rbase = pl.multiple_of(sid * rows_per_tile, 8)
        pltpu.sync_copy(zeros_hbm.at[pl.ds(0, rows_per_tile)],
                        acc_sh.at[pl.ds(rbase, rows_per_tile)])
        plsc.subcore_barrier()

        base = pl.multiple_of(wid * per_w, 8)

        def body(j, _):
            off = pl.multiple_of(base + j * CHUNK, 8)
            pltpu.sync_copy(src_hbm.at[pl.ds(off, CHUNK)], sidx_v)
            pltpu.sync_copy(dst_hbm.at[pl.ds(off, CHUNK)], didx_v)
            pltpu.async_copy(rows_hbm.at[sidx_v], rows_v, sem).wait()
            pltpu.sync_copy(rows_v, acc_sh.at[didx_v], add=True)
            return 0

        lax.fori_loop(0, nchunks, body, 0)
        plsc.subcore_barrier()
        pltpu.sync_copy(acc_sh.at[pl.ds(rbase, rows_per_tile)],
                        out_hbm.at[cid, pl.ds(rbase, rows_per_tile)])

    return k(rows, src, dst, zeros_rows)


BN = 80
BR = BN * TOK // 2
NROW = NTOK // 2
NBLK = N // BN


def _group_roll(arr, o, pos):
    t = arr.shape[0]
    a = jnp.concatenate([arr[o:], arr[:o]], axis=0)
    r2 = t + o - WALK_LEN
    b = jnp.concatenate([arr[r2:], arr[:r2]], axis=0)
    return jnp.where(pos < WALK_LEN - o, a, b)


def _t2_body(za_ref, zb_ref, deg_ref, hist_ref, wq_ref, wk_ref, wv_ref,
             wo_ref, w1_ref, w2_ref, e2_ref, e2t_ref, wse_ref, g1_ref,
             hn1_ref, dinv_ref):
    f32 = jnp.float32
    z = jnp.concatenate([za_ref[...][:, :HID], zb_ref[...][:, :HID]], axis=1)
    q = jnp.dot(z, wq_ref[...], preferred_element_type=f32) * (1.0 / 4.0)
    kk = jnp.dot(z, wk_ref[...], preferred_element_type=f32)
    v = jnp.dot(z, wv_ref[...], preferred_element_type=f32)

    E2 = e2_ref[...]
    E2T = e2t_ref[...]
    pos = lax.broadcasted_iota(jnp.int32, (BR, FW), 0) % WALK_LEN

    ks = [kk] + [_group_roll(kk, o, pos) for o in range(1, WALK_LEN)]
    logits = [jnp.dot(q * ko, E2, preferred_element_type=f32) for ko in ks]
    lcat = jnp.concatenate(logits, axis=1)
    e = jnp.exp(lcat)
    s8 = (lax.broadcasted_iota(jnp.int32, (8 * WALK_LEN, 8), 0) % 8 ==
          lax.broadcasted_iota(jnp.int32, (8 * WALK_LEN, 8), 1)).astype(f32)
    den8 = jnp.dot(e, s8, preferred_element_type=f32)
    inv_full = jnp.dot(1.0 / den8, E2T,
                       preferred_element_type=f32)
    o_acc = None
    for o in range(WALK_LEN):
        e_full = jnp.dot(e[:, 8 * o:8 * o + 8], E2T,
                         preferred_element_type=f32)
        vo = v if o == 0 else _group_roll(v, o, pos)
        contrib = e_full * vo
        o_acc = contrib if o_acc is None else o_acc + contrib
    o_acc = o_acc * inv_full
    z = z + jnp.dot(o_acc, wo_ref[...], preferred_element_type=f32)
    h1 = jnp.maximum(jnp.dot(z, w1_ref[...], preferred_element_type=f32), 0.0)
    z = z + jnp.dot(h1, w2_ref[...], preferred_element_type=f32)

    ri = lax.broadcasted_iota(jnp.int32, (BN, BR), 0)
    ci = lax.broadcasted_iota(jnp.int32, (BN, BR), 1) // (TOK // 2)
    P = jnp.where(ri == ci, 1.0 / TOK, 0.0).astype(f32)
    p128 = jnp.dot(P, z, preferred_element_type=f32)
    pooled = p128[:, :HID] + p128[:, HID:]

    deg = deg_ref[...]
    gf = 1.0 + jnp.log1p(jnp.maximum(deg, 0.0)) * wse_ref[...]
    gt = jnp.maximum(pooled * gf, 0.0)

    h = jnp.dot(gt, g1_ref[...], preferred_element_type=f32)
    degc = hist_ref[..., 0:1] + hist_ref[..., 1:2] + 1.0
    dinv = lax.rsqrt(degc)
    hn1_ref[...] = h * dinv
    dinv_ref[...] = dinv


def _bd(w):
    a, b = w.shape
    z = jnp.zeros((a, b), w.dtype)
    return jnp.concatenate([
        jnp.concatenate([w, z], axis=1),
        jnp.concatenate([z, w], axis=1),
    ], axis=0)


def _transformer(z, deg2, hist2, wq, wk, wv, wo, w1, w2, wse2, gcn1_w):
    f32 = jnp.float32
    bd_wq, bd_wk, bd_wv, bd_wo = _bd(wq), _bd(wk), _bd(wv), _bd(wo)
    bd_w1, bd_w2 = _bd(w1), _bd(w2)
    li = jnp.arange(FW)
    hj = jnp.arange(2 * HEADS)
    e2 = (li[:, None] // DH == hj[None, :]).astype(f32)
    e2t = e2.T
    wcon = pl.BlockSpec((FW, FW), lambda i: (0, 0))
    return pl.pallas_call(
        _t2_body,
        grid=(NBLK,),
        in_specs=[
            pl.BlockSpec((BR, FW), lambda i: (i, 0)),
            pl.BlockSpec((BR, FW), lambda i: (i + NBLK, 0)),
            pl.BlockSpec((BN, 1), lambda i: (i, 0)),
            pl.BlockSpec((BN, 2), lambda i: (i, 0)),
            wcon, wcon, wcon, wcon,
            pl.BlockSpec((FW, 2 * FW), lambda i: (0, 0)),
            pl.BlockSpec((2 * FW, FW), lambda i: (0, 0)),
            pl.BlockSpec((FW, 2 * HEADS), lambda i: (0, 0)),
            pl.BlockSpec((2 * HEADS, FW), lambda i: (0, 0)),
            pl.BlockSpec((1, HID), lambda i: (0, 0)),
            pl.BlockSpec((HID, 2 * OUT), lambda i: (0, 0)),
        ],
        out_specs=[
            pl.BlockSpec((BN, 2 * OUT), lambda i: (i, 0)),
            pl.BlockSpec((BN, 1), lambda i: (i, 0)),
        ],
        out_shape=[
            jax.ShapeDtypeStruct((N, 2 * OUT), jnp.float32),
            jax.ShapeDtypeStruct((N, 1), jnp.float32),
        ],
    )(z, z, deg2, hist2, bd_wq, bd_wk, bd_wv, bd_wo, bd_w1, bd_w2,
      e2, e2t, wse2, gcn1_w)


def _t3_body(p0_ref, p1_ref, hn1_ref, dinv_ref, b1_ref, g2_ref, hn2_ref):
    dinv = dinv_ref[...]
    s = p0_ref[...] + p1_ref[...] + hn1_ref[...]
    h1 = jnp.maximum(dinv * s + b1_ref[...], 0.0)
    hn2_ref[...] = dinv * jnp.dot(h1, g2_ref[...],
                                  preferred_element_type=jnp.float32)


def _gcn_mid(p0, p1, hn1, dinv, b1_2, gcn2_w_pad):
    blk = 2000
    return pl.pallas_call(
        _t3_body,
        grid=(N // blk,),
        in_specs=[
            pl.BlockSpec((blk, FW), lambda i: (i, 0)),
            pl.BlockSpec((blk, FW), lambda i: (i, 0)),
            pl.BlockSpec((blk, 2 * OUT), lambda i: (i, 0)),
            pl.BlockSpec((blk, 1), lambda i: (i, 0)),
            pl.BlockSpec((1, 2 * OUT), lambda i: (0, 0)),
            pl.BlockSpec((2 * OUT, FW), lambda i: (0, 0)),
        ],
        out_specs=pl.BlockSpec((blk, FW), lambda i: (i, 0)),
        out_shape=jax.ShapeDtypeStruct((N, FW), jnp.float32),
    )(p0, p1, hn1, dinv, b1_2, gcn2_w_pad)


def _t4_body(p0_ref, p1_ref, hn2_ref, dinv_ref, b2_ref, o_ref):
    s = p0_ref[...] + p1_ref[...] + hn2_ref[...]
    s = dinv_ref[...] * s[:, :OUT] + b2_ref[...]
    m = jnp.max(s, axis=1, keepdims=True)
    e = jnp.exp(s - m)
    o_ref[...] = e / jnp.sum(e, axis=1, keepdims=True)


def _finalize(p0, p1, hn2, dinv, b2_2):
    blk = 2000
    return pl.pallas_call(
        _t4_body,
        grid=(N // blk,),
        in_specs=[
            pl.BlockSpec((blk, FW), lambda i: (i, 0)),
            pl.BlockSpec((blk, FW), lambda i: (i, 0)),
            pl.BlockSpec((blk, FW), lambda i: (i, 0)),
            pl.BlockSpec((blk, 1), lambda i: (i, 0)),
            pl.BlockSpec((1, OUT), lambda i: (0, 0)),
        ],
        out_specs=pl.BlockSpec((blk, OUT), lambda i: (i, 0)),
        out_shape=jax.ShapeDtypeStruct((N, OUT), jnp.float32),
    )(p0, p1, hn2, dinv, b2_2)


def kernel(x, deg, edge_index, walks, w_in, wq, wk, wv, wo, w1, w2, w_se,
           gcn1_w, gcn1_b, gcn2_w, gcn2_b):
    f32 = jnp.float32
    wflat = jnp.concatenate([
        walks[:, :2, :].reshape(-1), walks[:, 2:, :].reshape(-1)
    ]).astype(jnp.int32)
    src = edge_index[0].astype(jnp.int32)
    dst = edge_index[1].astype(jnp.int32)

    ones_rows = jnp.ones((CHUNK, FW), f32)
    zeros_rows = jnp.zeros((NPAD // NS, FW), f32)

    w_in_pad = jnp.pad(w_in, ((0, 0), (0, FW - HID)))
    xp = _project(x, w_in_pad)
    z, hist = _gather_and_hist(xp, wflat, dst, ones_rows, zeros_rows)
    hist2 = hist[:, :N, 0].T

    hn1, dinv = _transformer(
        z, deg.reshape(N, 1), hist2, wq[0], wk[0], wv[0], wo[0],
        w1[0], w2[0], w_se.reshape(1, HID), gcn1_w)

    s1 = _edge_scatter(hn1, src, dst, zeros_rows)
    gcn2_w_pad = jnp.pad(gcn2_w, ((0, 0), (0, FW - OUT)))
    hn2 = _gcn_mid(s1[0, :N], s1[1, :N], hn1, dinv,
                   gcn1_b.reshape(1, 2 * OUT), gcn2_w_pad)

    s2 = _edge_scatter(hn2, src, dst, zeros_rows)
    return _finalize(s2[0, :N], s2[1, :N], hn2, dinv,
                     gcn2_b.reshape(1, OUT))


# speedup vs baseline: 1.9155x; 1.2413x over previous
import functools

import jax
import jax.numpy as jnp
from jax import lax
from jax.experimental import pallas as pl
from jax.experimental.pallas import tpu as pltpu
from jax.experimental.pallas import tpu_sc as plsc

N = 10000
IN_DIM = 128
HID = 64
OUT = 64
NUM_WALKS = 4
WALK_LEN = 8
HEADS = 4
DH = HID // HEADS
N_EDGES = 320000
TOK = NUM_WALKS * WALK_LEN
NTOK = N * TOK
FW = 128

NPAD = 10240
NC, NS = 2, 16
NW = NC * NS
CHUNK = 80


def _t1_body(x_ref, w_ref, o_ref):
    o_ref[...] = jnp.dot(x_ref[...], w_ref[...],
                         preferred_element_type=jnp.float32)


def _project(x, w_in_pad):
    blk = 2000
    return pl.pallas_call(
        _t1_body,
        grid=(N // blk,),
        in_specs=[
            pl.BlockSpec((blk, IN_DIM), lambda i: (i, 0)),
            pl.BlockSpec((IN_DIM, FW), lambda i: (0, 0)),
        ],
        out_specs=pl.BlockSpec((blk, FW), lambda i: (i, 0)),
        out_shape=jax.ShapeDtypeStruct((N, FW), jnp.float32),
    )(x, w_in_pad)


def _gather_and_hist(xp, wflat, dst, ones_rows, zeros_rows):
    per_w = NTOK // NW
    nchunks = per_w // CHUNK
    rows_per_tile = NPAD // NS

    mesh = plsc.VectorSubcoreMesh(core_axis_name="c", subcore_axis_name="s")

    @functools.partial(
        pl.kernel, mesh=mesh,
        out_type=[
            jax.ShapeDtypeStruct((NTOK, FW), jnp.float32),
            jax.ShapeDtypeStruct((NC, NPAD, FW), jnp.float32),
        ],
        scratch_types=[
            pltpu.VMEM((CHUNK,), jnp.int32),
            pltpu.VMEM((CHUNK,), jnp.int32),
            pltpu.VMEM((CHUNK,), jnp.int32),
            pltpu.VMEM((CHUNK,), jnp.int32),
            pltpu.VMEM((CHUNK, FW), jnp.float32),
            pltpu.VMEM((CHUNK, FW), jnp.float32),
            pltpu.VMEM((CHUNK, FW), jnp.float32),
            pltpu.VMEM_SHARED((NPAD, FW), jnp.float32),
            pltpu.SemaphoreType.DMA,
            pltpu.SemaphoreType.DMA,
        ],
    )
    def k(xp_hbm, idx_hbm, dst_hbm, ones_hbm, zeros_hbm, z_hbm, hist_hbm,
          i0, i1, d0, d1, r0, r1, ones_v, acc_sh, sem0, sem1):
        cid = lax.axis_index("c")
        sid = lax.axis_index("s")
        wid = cid * NS + sid
        rbase = pl.multiple_of(sid * rows_per_tile, 8)
        pltpu.sync_copy(zeros_hbm.at[pl.ds(0, rows_per_tile)],
                        acc_sh.at[pl.ds(rbase, rows_per_tile)])
        pltpu.sync_copy(ones_hbm, ones_v)
        plsc.subcore_barrier()

        base = pl.multiple_of(wid * per_w, 8)

        def off(c):
            return pl.multiple_of(base + c * CHUNK, 8)

        def load(c, iv, dv):
            pltpu.sync_copy(idx_hbm.at[pl.ds(off(c), CHUNK)], iv)
            pltpu.sync_copy(dst_hbm.at[pl.ds(off(c), CHUNK)], dv)

        def consume(c, iv, dv, rv, sem):
            pltpu.make_async_copy(xp_hbm.at[iv], rv, sem).wait()
            pltpu.sync_copy(rv, z_hbm.at[pl.ds(off(c), CHUNK)])
            pltpu.sync_copy(ones_v, acc_sh.at[dv], add=True)

        load(0, i0, d0)
        pltpu.async_copy(xp_hbm.at[i0], r0, sem0)

        def body(j, _):
            c0 = 2 * j
            load(c0 + 1, i1, d1)
            pltpu.async_copy(xp_hbm.at[i1], r1, sem1)
            consume(c0, i0, d0, r0, sem0)
            load(c0 + 2, i0, d0)
            pltpu.async_copy(xp_hbm.at[i0], r0, sem0)
            consume(c0 + 1, i1, d1, r1, sem1)
            return 0

        lax.fori_loop(0, (nchunks - 1) // 2, body, 0)
        consume(nchunks - 1, i0, d0, r0, sem0)
        plsc.subcore_barrier()
        pltpu.sync_copy(acc_sh.at[pl.ds(rbase, rows_per_tile)],
                        hist_hbm.at[cid, pl.ds(rbase, rows_per_tile)])

    return k(xp, wflat, dst, ones_rows, zeros_rows)


def _edge_scatter(rows, src, dst, zeros_rows):
    per_w = N_EDGES // NW
    nchunks = per_w // CHUNK
    rows_per_tile = NPAD // NS

    mesh = plsc.VectorSubcoreMesh(core_axis_name="c", subcore_axis_name="s")

    @functools.partial(
        pl.kernel, mesh=mesh,
        out_type=jax.ShapeDtypeStruct((NC, NPAD, FW), jnp.float32),
        scratch_types=[
            pltpu.VMEM((CHUNK,), jnp.int32),
            pltpu.VMEM((CHUNK,), jnp.int32),
            pltpu.VMEM((CHUNK,), jnp.int32),
            pltpu.VMEM((CHUNK,), jnp.int32),
            pltpu.VMEM((CHUNK, FW), jnp.float32),
            pltpu.VMEM((CHUNK, FW), jnp.float32),
            pltpu.VMEM_SHARED((NPAD, FW), jnp.float32),
            pltpu.SemaphoreType.DMA,
            pltpu.SemaphoreType.DMA,
        ],
    )
    def k(rows_hbm, src_hbm, dst_hbm, zeros_hbm, out_hbm,
          s0, s1, d0, d1, r0, r1, acc_sh, sem0, sem1):
        cid = lax.axis_index("c")
        sid = lax.axis_index("s")
        wid = cid * NS + sid
        rbase = pl.multiple_of(sid * rows_per_tile, 8)
        pltpu.sync_copy(zeros_hbm.at[pl.ds(0, rows_per_tile)],
                        acc_sh.at[pl.ds(rbase, rows_per_tile)])
        plsc.subcore_barrier()

        base = pl.multiple_of(wid * per_w, 8)

        def off(c):
            return pl.multiple_of(base + c * CHUNK, 8)

        def load(c, sv, dv):
            pltpu.sync_copy(src_hbm.at[pl.ds(off(c), CHUNK)], sv)
            pltpu.sync_copy(dst_hbm.at[pl.ds(off(c), CHUNK)], dv)

        def consume(sv, dv, rv, sem):
            pltpu.make_async_copy(rows_hbm.at[sv], rv, sem).wait()
            pltpu.sync_copy(rv, acc_sh.at[dv], add=True)

        load(0, s0, d0)
        pltpu.async_copy(rows_hbm.at[s0], r0, sem0)

        def body(j, _):
            c0 = 2 * j
            load(c0 + 1, s1, d1)
            pltpu.async_copy(rows_hbm.at[s1], r1, sem1)
            consume(s0, d0, r0, sem0)
            load(c0 + 2, s0, d0)
            pltpu.async_copy(rows_hbm.at[s0], r0, sem0)
            consume(s1, d1, r1, sem1)
            return 0

        lax.fori_loop(0, (nchunks - 1) // 2, body, 0)
        consume(s0, d0, r0, sem0)
        plsc.subcore_barrier()
        pltpu.sync_copy(acc_sh.at[pl.ds(rbase, rows_per_tile)],
                        out_hbm.at[cid, pl.ds(rbase, rows_per_tile)])

    return k(rows, src, dst, zeros_rows)


BN = 80
BR = BN * TOK // 2
NROW = NTOK // 2
NBLK = N // BN


def _group_roll(arr, o, pos):
    t = arr.shape[0]
    a = jnp.concatenate([arr[o:], arr[:o]], axis=0)
    r2 = t + o - WALK_LEN
    b = jnp.concatenate([arr[r2:], arr[:r2]], axis=0)
    return jnp.where(pos < WALK_LEN - o, a, b)


def _t2_body(za_ref, zb_ref, deg_ref, hist_ref, wq_ref, wk_ref, wv_ref,
             wo_ref, w1_ref, w2_ref, e2_ref, e2t_ref, wse_ref, g1_ref,
             hn1_ref, dinv_ref):
    f32 = jnp.float32
    z = jnp.concatenate([za_ref[...][:, :HID], zb_ref[...][:, :HID]], axis=1)
    q = jnp.dot(z, wq_ref[...], preferred_element_type=f32) * (1.0 / 4.0)
    kk = jnp.dot(z, wk_ref[...], preferred_element_type=f32)
    v = jnp.dot(z, wv_ref[...], preferred_element_type=f32)

    E2 = e2_ref[...]
    E2T = e2t_ref[...]
    pos = lax.broadcasted_iota(jnp.int32, (BR, FW), 0) % WALK_LEN

    ks = [kk] + [_group_roll(kk, o, pos) for o in range(1, WALK_LEN)]
    logits = [jnp.dot(q * ko, E2, preferred_element_type=f32) for ko in ks]
    lcat = jnp.concatenate(logits, axis=1)
    e = jnp.exp(lcat)
    s8 = (lax.broadcasted_iota(jnp.int32, (8 * WALK_LEN, 8), 0) % 8 ==
          lax.broadcasted_iota(jnp.int32, (8 * WALK_LEN, 8), 1)).astype(f32)
    den8 = jnp.dot(e, s8, preferred_element_type=f32)
    inv_full = jnp.dot(1.0 / den8, E2T,
                       preferred_element_type=f32)
    o_acc = None
    for o in range(WALK_LEN):
        e_full = jnp.dot(e[:, 8 * o:8 * o + 8], E2T,
                         preferred_element_type=f32)
        vo = v if o == 0 else _group_roll(v, o, pos)
        contrib = e_full * vo
        o_acc = contrib if o_acc is None else o_acc + contrib
    o_acc = o_acc * inv_full
    z = z + jnp.dot(o_acc, wo_ref[...], preferred_element_type=f32)
    h1 = jnp.maximum(jnp.dot(z, w1_ref[...], preferred_element_type=f32), 0.0)
    z = z + jnp.dot(h1, w2_ref[...], preferred_element_type=f32)

    ri = lax.broadcasted_iota(jnp.int32, (BN, BR), 0)
    ci = lax.broadcasted_iota(jnp.int32, (BN, BR), 1) // (TOK // 2)
    P = jnp.where(ri == ci, 1.0 / TOK, 0.0).astype(f32)
    p128 = jnp.dot(P, z, preferred_element_type=f32)
    pooled = p128[:, :HID] + p128[:, HID:]

    deg = deg_ref[...]
    gf = 1.0 + jnp.log1p(jnp.maximum(deg, 0.0)) * wse_ref[...]
    gt = jnp.maximum(pooled * gf, 0.0)

    h = jnp.dot(gt, g1_ref[...], preferred_element_type=f32)
    degc = hist_ref[..., 0:1] + hist_ref[..., 1:2] + 1.0
    dinv = lax.rsqrt(degc)
    hn1_ref[...] = h * dinv
    dinv_ref[...] = dinv


def _bd(w):
    a, b = w.shape
    z = jnp.zeros((a, b), w.dtype)
    return jnp.concatenate([
        jnp.concatenate([w, z], axis=1),
        jnp.concatenate([z, w], axis=1),
    ], axis=0)


def _transformer(z, deg2, hist2, wq, wk, wv, wo, w1, w2, wse2, gcn1_w):
    f32 = jnp.float32
    bd_wq, bd_wk, bd_wv, bd_wo = _bd(wq), _bd(wk), _bd(wv), _bd(wo)
    bd_w1, bd_w2 = _bd(w1), _bd(w2)
    li = jnp.arange(FW)
    hj = jnp.arange(2 * HEADS)
    e2 = (li[:, None] // DH == hj[None, :]).astype(f32)
    e2t = e2.T
    wcon = pl.BlockSpec((FW, FW), lambda i: (0, 0))
    return pl.pallas_call(
        _t2_body,
        grid=(NBLK,),
        in_specs=[
            pl.BlockSpec((BR, FW), lambda i: (i, 0)),
            pl.BlockSpec((BR, FW), lambda i: (i + NBLK, 0)),
            pl.BlockSpec((BN, 1), lambda i: (i, 0)),
            pl.BlockSpec((BN, 2), lambda i: (i, 0)),
            wcon, wcon, wcon, wcon,
            pl.BlockSpec((FW, 2 * FW), lambda i: (0, 0)),
            pl.BlockSpec((2 * FW, FW), lambda i: (0, 0)),
            pl.BlockSpec((FW, 2 * HEADS), lambda i: (0, 0)),
            pl.BlockSpec((2 * HEADS, FW), lambda i: (0, 0)),
            pl.BlockSpec((1, HID), lambda i: (0, 0)),
            pl.BlockSpec((HID, 2 * OUT), lambda i: (0, 0)),
        ],
        out_specs=[
            pl.BlockSpec((BN, 2 * OUT), lambda i: (i, 0)),
            pl.BlockSpec((BN, 1), lambda i: (i, 0)),
        ],
        out_shape=[
            jax.ShapeDtypeStruct((N, 2 * OUT), jnp.float32),
            jax.ShapeDtypeStruct((N, 1), jnp.float32),
        ],
    )(z, z, deg2, hist2, bd_wq, bd_wk, bd_wv, bd_wo, bd_w1, bd_w2,
      e2, e2t, wse2, gcn1_w)


def _t3_body(p0_ref, p1_ref, hn1_ref, dinv_ref, b1_ref, g2_ref, hn2_ref):
    dinv = dinv_ref[...]
    s = p0_ref[...] + p1_ref[...] + hn1_ref[...]
    h1 = jnp.maximum(dinv * s + b1_ref[...], 0.0)
    hn2_ref[...] = dinv * jnp.dot(h1, g2_ref[...],
                                  preferred_element_type=jnp.float32)


def _gcn_mid(p0, p1, hn1, dinv, b1_2, gcn2_w_pad):
    blk = 2000
    return pl.pallas_call(
        _t3_body,
        grid=(N // blk,),
        in_specs=[
            pl.BlockSpec((blk, FW), lambda i: (i, 0)),
            pl.BlockSpec((blk, FW), lambda i: (i, 0)),
            pl.BlockSpec((blk, 2 * OUT), lambda i: (i, 0)),
            pl.BlockSpec((blk, 1), lambda i: (i, 0)),
            pl.BlockSpec((1, 2 * OUT), lambda i: (0, 0)),
            pl.BlockSpec((2 * OUT, FW), lambda i: (0, 0)),
        ],
        out_specs=pl.BlockSpec((blk, FW), lambda i: (i, 0)),
        out_shape=jax.ShapeDtypeStruct((N, FW), jnp.float32),
    )(p0, p1, hn1, dinv, b1_2, gcn2_w_pad)


def _t4_body(p0_ref, p1_ref, hn2_ref, dinv_ref, b2_ref, o_ref):
    s = p0_ref[...] + p1_ref[...] + hn2_ref[...]
    s = dinv_ref[...] * s[:, :OUT] + b2_ref[...]
    m = jnp.max(s, axis=1, keepdims=True)
    e = jnp.exp(s - m)
    o_ref[...] = e / jnp.sum(e, axis=1, keepdims=True)


def _finalize(p0, p1, hn2, dinv, b2_2):
    blk = 2000
    return pl.pallas_call(
        _t4_body,
        grid=(N // blk,),
        in_specs=[
            pl.BlockSpec((blk, FW), lambda i: (i, 0)),
            pl.BlockSpec((blk, FW), lambda i: (i, 0)),
            pl.BlockSpec((blk, FW), lambda i: (i, 0)),
            pl.BlockSpec((blk, 1), lambda i: (i, 0)),
            pl.BlockSpec((1, OUT), lambda i: (0, 0)),
        ],
        out_specs=pl.BlockSpec((blk, OUT), lambda i: (i, 0)),
        out_shape=jax.ShapeDtypeStruct((N, OUT), jnp.float32),
    )(p0, p1, hn2, dinv, b2_2)


def kernel(x, deg, edge_index, walks, w_in, wq, wk, wv, wo, w1, w2, w_se,
           gcn1_w, gcn1_b, gcn2_w, gcn2_b):
    f32 = jnp.float32
    wflat = jnp.concatenate([
        walks[:, :2, :].reshape(-1), walks[:, 2:, :].reshape(-1)
    ]).astype(jnp.int32)
    src = edge_index[0].astype(jnp.int32)
    dst = edge_index[1].astype(jnp.int32)

    ones_rows = jnp.ones((CHUNK, FW), f32)
    zeros_rows = jnp.zeros((NPAD // NS, FW), f32)

    w_in_pad = jnp.pad(w_in, ((0, 0), (0, FW - HID)))
    xp = _project(x, w_in_pad)
    z, hist = _gather_and_hist(xp, wflat, dst, ones_rows, zeros_rows)
    hist2 = hist[:, :N, 0].T

    hn1, dinv = _transformer(
        z, deg.reshape(N, 1), hist2, wq[0], wk[0], wv[0], wo[0],
        w1[0], w2[0], w_se.reshape(1, HID), gcn1_w)

    s1 = _edge_scatter(hn1, src, dst, zeros_rows)
    gcn2_w_pad = jnp.pad(gcn2_w, ((0, 0), (0, FW - OUT)))
    hn2 = _gcn_mid(s1[0, :N], s1[1, :N], hn1, dinv,
                   gcn1_b.reshape(1, 2 * OUT), gcn2_w_pad)

    s2 = _edge_scatter(hn2, src, dst, zeros_rows)
    return _finalize(s2[0, :N], s2[1, :N], hn2, dinv,
                     gcn2_b.reshape(1, OUT))


# speedup vs baseline: 2.4243x; 1.2656x over previous
import functools

import jax
import jax.numpy as jnp
from jax import lax
from jax.experimental import pallas as pl
from jax.experimental.pallas import tpu as pltpu
from jax.experimental.pallas import tpu_sc as plsc

N = 10000
IN_DIM = 128
HID = 64
OUT = 64
NUM_WALKS = 4
WALK_LEN = 8
HEADS = 4
DH = HID // HEADS
N_EDGES = 320000
TOK = NUM_WALKS * WALK_LEN
NTOK = N * TOK
FW = 128

NPAD = 10240
NC, NS = 2, 16
NW = NC * NS
CHUNK = 80


def _t1_body(x_ref, w_ref, o_ref):
    o_ref[...] = jnp.dot(x_ref[...], w_ref[...],
                         preferred_element_type=jnp.float32)


def _project(x, w_in_pad):
    blk = 2000
    return pl.pallas_call(
        _t1_body,
        grid=(N // blk,),
        in_specs=[
            pl.BlockSpec((blk, IN_DIM), lambda i: (i, 0)),
            pl.BlockSpec((IN_DIM, FW), lambda i: (0, 0)),
        ],
        out_specs=pl.BlockSpec((blk, FW), lambda i: (i, 0)),
        out_shape=jax.ShapeDtypeStruct((N, FW), jnp.float32),
    )(x, w_in_pad)


def _gather_rows(xp, wflat):
    per_w = NTOK // NW
    nchunks = per_w // CHUNK
    rows_per_tile = NPAD // NS

    mesh = plsc.VectorSubcoreMesh(core_axis_name="c", subcore_axis_name="s")

    @functools.partial(
        pl.kernel, mesh=mesh,
        out_type=jax.ShapeDtypeStruct((NTOK, FW), jnp.float32),
        scratch_types=[
            pltpu.VMEM((CHUNK,), jnp.int32),
            pltpu.VMEM((CHUNK,), jnp.int32),
            pltpu.VMEM((CHUNK, FW), jnp.float32),
            pltpu.VMEM((CHUNK, FW), jnp.float32),
            pltpu.SemaphoreType.DMA,
            pltpu.SemaphoreType.DMA,
        ],
    )
    def k(xp_hbm, idx_hbm, z_hbm, i0, i1, r0, r1, sem0, sem1):
        cid = lax.axis_index("c")
        sid = lax.axis_index("s")
        wid = cid * NS + sid
        base = pl.multiple_of(wid * per_w, 8)

        def off(c):
            return pl.multiple_of(base + c * CHUNK, 8)

        def load(c, iv):
            pltpu.sync_copy(idx_hbm.at[pl.ds(off(c), CHUNK)], iv)

        def consume(c, iv, rv, sem):
            pltpu.make_async_copy(xp_hbm.at[iv], rv, sem).wait()
            pltpu.sync_copy(rv, z_hbm.at[pl.ds(off(c), CHUNK)])

        load(0, i0)
        pltpu.async_copy(xp_hbm.at[i0], r0, sem0)

        def body(j, _):
            c0 = 2 * j
            load(c0 + 1, i1)
            pltpu.async_copy(xp_hbm.at[i1], r1, sem1)
            consume(c0, i0, r0, sem0)
            load(c0 + 2, i0)
            pltpu.async_copy(xp_hbm.at[i0], r0, sem0)
            consume(c0 + 1, i1, r1, sem1)
            return 0

        lax.fori_loop(0, (nchunks - 1) // 2, body, 0)
        consume(nchunks - 1, i0, r0, sem0)

    return k(xp, wflat)


def _dst_histogram(dst, ones_rows, zeros_rows):
    per_w = N_EDGES // NW
    nchunks = per_w // CHUNK
    rows_per_tile = NPAD // NS

    mesh = plsc.VectorSubcoreMesh(core_axis_name="c", subcore_axis_name="s")

    @functools.partial(
        pl.kernel, mesh=mesh,
        out_type=jax.ShapeDtypeStruct((NC, NPAD, FW), jnp.float32),
        scratch_types=[
            pltpu.VMEM((CHUNK,), jnp.int32),
            pltpu.VMEM((CHUNK, FW), jnp.float32),
            pltpu.VMEM_SHARED((NPAD, FW), jnp.float32),
        ],
    )
    def k(dst_hbm, ones_hbm, zeros_hbm, hist_hbm, didx_v, ones_v, acc_sh):
        cid = lax.axis_index("c")
        sid = lax.axis_index("s")
        wid = cid * NS + sid
        rbase = pl.multiple_of(sid * rows_per_tile, 8)
        pltpu.sync_copy(zeros_hbm.at[pl.ds(0, rows_per_tile)],
                        acc_sh.at[pl.ds(rbase, rows_per_tile)])
        pltpu.sync_copy(ones_hbm, ones_v)
        plsc.subcore_barrier()

        base = pl.multiple_of(wid * per_w, 8)

        def body(j, _):
            off = pl.multiple_of(base + j * CHUNK, 8)
            pltpu.sync_copy(dst_hbm.at[pl.ds(off, CHUNK)], didx_v)
            pltpu.sync_copy(ones_v, acc_sh.at[didx_v], add=True)
            return 0

        lax.fori_loop(0, nchunks, body, 0)
        plsc.subcore_barrier()
        pltpu.sync_copy(acc_sh.at[pl.ds(rbase, rows_per_tile)],
                        hist_hbm.at[cid, pl.ds(rbase, rows_per_tile)])

    return k(dst, ones_rows, zeros_rows)


def _edge_scatter(rows, src, dst, zeros_rows):
    per_w = N_EDGES // NW
    nchunks = per_w // CHUNK
    rows_per_tile = NPAD // NS

    mesh = plsc.VectorSubcoreMesh(core_axis_name="c", subcore_axis_name="s")

    @functools.partial(
        pl.kernel, mesh=mesh,
        out_type=jax.ShapeDtypeStruct((NC, NPAD, FW), jnp.float32),
        scratch_types=[
            pltpu.VMEM((CHUNK,), jnp.int32),
            pltpu.VMEM((CHUNK,), jnp.int32),
            pltpu.VMEM((CHUNK,), jnp.int32),
            pltpu.VMEM((CHUNK,), jnp.int32),
            pltpu.VMEM((CHUNK, FW), jnp.float32),
            pltpu.VMEM((CHUNK, FW), jnp.float32),
            pltpu.VMEM_SHARED((NPAD, FW), jnp.float32),
            pltpu.SemaphoreType.DMA,
            pltpu.SemaphoreType.DMA,
        ],
    )
    def k(rows_hbm, src_hbm, dst_hbm, zeros_hbm, out_hbm,
          s0, s1, d0, d1, r0, r1, acc_sh, sem0, sem1):
        cid = lax.axis_index("c")
        sid = lax.axis_index("s")
        wid = cid * NS + sid
        rbase = pl.multiple_of(sid * rows_per_tile, 8)
        pltpu.sync_copy(zeros_hbm.at[pl.ds(0, rows_per_tile)],
                        acc_sh.at[pl.ds(rbase, rows_per_tile)])
        plsc.subcore_barrier()

        base = pl.multiple_of(wid * per_w, 8)

        def off(c):
            return pl.multiple_of(base + c * CHUNK, 8)

        def load(c, sv, dv):
            pltpu.sync_copy(src_hbm.at[pl.ds(off(c), CHUNK)], sv)
            pltpu.sync_copy(dst_hbm.at[pl.ds(off(c), CHUNK)], dv)

        def consume(sv, dv, rv, sem):
            pltpu.make_async_copy(rows_hbm.at[sv], rv, sem).wait()
            pltpu.sync_copy(rv, acc_sh.at[dv], add=True)

        load(0, s0, d0)
        pltpu.async_copy(rows_hbm.at[s0], r0, sem0)

        def body(j, _):
            c0 = 2 * j
            load(c0 + 1, s1, d1)
            pltpu.async_copy(rows_hbm.at[s1], r1, sem1)
            consume(s0, d0, r0, sem0)
            load(c0 + 2, s0, d0)
            pltpu.async_copy(rows_hbm.at[s0], r0, sem0)
            consume(s1, d1, r1, sem1)
            return 0

        lax.fori_loop(0, (nchunks - 1) // 2, body, 0)
        consume(s0, d0, r0, sem0)
        plsc.subcore_barrier()
        pltpu.sync_copy(acc_sh.at[pl.ds(rbase, rows_per_tile)],
                        out_hbm.at[cid, pl.ds(rbase, rows_per_tile)])

    return k(rows, src, dst, zeros_rows)


BN = 80
BR = BN * TOK // 2
NROW = NTOK // 2
NBLK = N // BN


def _group_roll(arr, o, pos):
    t = arr.shape[0]
    a = jnp.concatenate([arr[o:], arr[:o]], axis=0)
    r2 = t + o - WALK_LEN
    b = jnp.concatenate([arr[r2:], arr[:r2]], axis=0)
    return jnp.where(pos < WALK_LEN - o, a, b)


def _t2_body(za_ref, zb_ref, deg_ref, wq_ref, wk_ref, wv_ref,
             wo_ref, w1_ref, w2_ref, e2_ref, e2t_ref, wse_ref, g1_ref,
             h_ref):
    f32 = jnp.float32
    z = jnp.concatenate([za_ref[...][:, :HID], zb_ref[...][:, :HID]], axis=1)
    q = jnp.dot(z, wq_ref[...], preferred_element_type=f32) * (1.0 / 4.0)
    kk = jnp.dot(z, wk_ref[...], preferred_element_type=f32)
    v = jnp.dot(z, wv_ref[...], preferred_element_type=f32)

    E2 = e2_ref[...]
    E2T = e2t_ref[...]
    pos = lax.broadcasted_iota(jnp.int32, (BR, FW), 0) % WALK_LEN

    ks = [kk] + [_group_roll(kk, o, pos) for o in range(1, WALK_LEN)]
    logits = [jnp.dot(q * ko, E2, preferred_element_type=f32) for ko in ks]
    lcat = jnp.concatenate(logits, axis=1)
    e = jnp.exp(lcat)
    s8 = (lax.broadcasted_iota(jnp.int32, (8 * WALK_LEN, 8), 0) % 8 ==
          lax.broadcasted_iota(jnp.int32, (8 * WALK_LEN, 8), 1)).astype(f32)
    den8 = jnp.dot(e, s8, preferred_element_type=f32)
    inv_full = jnp.dot(1.0 / den8, E2T,
                       preferred_element_type=f32)
    o_acc = None
    for o in range(WALK_LEN):
        e_full = jnp.dot(e[:, 8 * o:8 * o + 8], E2T,
                         preferred_element_type=f32)
        vo = v if o == 0 else _group_roll(v, o, pos)
        contrib = e_full * vo
        o_acc = contrib if o_acc is None else o_acc + contrib
    o_acc = o_acc * inv_full
    z = z + jnp.dot(o_acc, wo_ref[...], preferred_element_type=f32)
    h1 = jnp.maximum(jnp.dot(z, w1_ref[...], preferred_element_type=f32), 0.0)
    z = z + jnp.dot(h1, w2_ref[...], preferred_element_type=f32)

    ri = lax.broadcasted_iota(jnp.int32, (BN, BR), 0)
    ci = lax.broadcasted_iota(jnp.int32, (BN, BR), 1) // (TOK // 2)
    P = jnp.where(ri == ci, 1.0 / TOK, 0.0).astype(f32)
    p128 = jnp.dot(P, z, preferred_element_type=f32)
    pooled = p128[:, :HID] + p128[:, HID:]

    deg = deg_ref[...]
    gf = 1.0 + jnp.log1p(jnp.maximum(deg, 0.0)) * wse_ref[...]
    gt = jnp.maximum(pooled * gf, 0.0)

    h_ref[...] = jnp.dot(gt, g1_ref[...], preferred_element_type=f32)


def _bd(w):
    a, b = w.shape
    z = jnp.zeros((a, b), w.dtype)
    return jnp.concatenate([
        jnp.concatenate([w, z], axis=1),
        jnp.concatenate([z, w], axis=1),
    ], axis=0)


def _transformer(z, deg2, wq, wk, wv, wo, w1, w2, wse2, gcn1_w):
    f32 = jnp.float32
    bd_wq, bd_wk, bd_wv, bd_wo = _bd(wq), _bd(wk), _bd(wv), _bd(wo)
    bd_w1, bd_w2 = _bd(w1), _bd(w2)
    li = jnp.arange(FW)
    hj = jnp.arange(2 * HEADS)
    e2 = (li[:, None] // DH == hj[None, :]).astype(f32)
    e2t = e2.T
    wcon = pl.BlockSpec((FW, FW), lambda i: (0, 0))
    return pl.pallas_call(
        _t2_body,
        grid=(NBLK,),
        in_specs=[
            pl.BlockSpec((BR, FW), lambda i: (i, 0)),
            pl.BlockSpec((BR, FW), lambda i: (i + NBLK, 0)),
            pl.BlockSpec((BN, 1), lambda i: (i, 0)),
            wcon, wcon, wcon, wcon,
            pl.BlockSpec((FW, 2 * FW), lambda i: (0, 0)),
            pl.BlockSpec((2 * FW, FW), lambda i: (0, 0)),
            pl.BlockSpec((FW, 2 * HEADS), lambda i: (0, 0)),
            pl.BlockSpec((2 * HEADS, FW), lambda i: (0, 0)),
            pl.BlockSpec((1, HID), lambda i: (0, 0)),
            pl.BlockSpec((HID, 2 * OUT), lambda i: (0, 0)),
        ],
        out_specs=pl.BlockSpec((BN, 2 * OUT), lambda i: (i, 0)),
        out_shape=jax.ShapeDtypeStruct((N, 2 * OUT), jnp.float32),
    )(z, z, deg2, bd_wq, bd_wk, bd_wv, bd_wo, bd_w1, bd_w2,
      e2, e2t, wse2, gcn1_w)


def _t2b_body(h_ref, h0_ref, h1_ref, hn1_ref, dinv_ref):
    degc = h0_ref[...][:, 0:1] + h1_ref[...][:, 0:1] + 1.0
    dinv = lax.rsqrt(degc)
    hn1_ref[...] = h_ref[...] * dinv
    dinv_ref[...] = dinv


def _apply_dinv(h, hist0, hist1):
    blk = 2000
    return pl.pallas_call(
        _t2b_body,
        grid=(N // blk,),
        in_specs=[
            pl.BlockSpec((blk, FW), lambda i: (i, 0)),
            pl.BlockSpec((blk, FW), lambda i: (i, 0)),
            pl.BlockSpec((blk, FW), lambda i: (i, 0)),
        ],
        out_specs=[
            pl.BlockSpec((blk, FW), lambda i: (i, 0)),
            pl.BlockSpec((blk, 1), lambda i: (i, 0)),
        ],
        out_shape=[
            jax.ShapeDtypeStruct((N, FW), jnp.float32),
            jax.ShapeDtypeStruct((N, 1), jnp.float32),
        ],
    )(h, hist0, hist1)


def _t3_body(p0_ref, p1_ref, hn1_ref, dinv_ref, b1_ref, g2_ref, hn2_ref):
    dinv = dinv_ref[...]
    s = p0_ref[...] + p1_ref[...] + hn1_ref[...]
    h1 = jnp.maximum(dinv * s + b1_ref[...], 0.0)
    hn2_ref[...] = dinv * jnp.dot(h1, g2_ref[...],
                                  preferred_element_type=jnp.float32)


def _gcn_mid(p0, p1, hn1, dinv, b1_2, gcn2_w_pad):
    blk = 2000
    return pl.pallas_call(
        _t3_body,
        grid=(N // blk,),
        in_specs=[
            pl.BlockSpec((blk, FW), lambda i: (i, 0)),
            pl.BlockSpec((blk, FW), lambda i: (i, 0)),
            pl.BlockSpec((blk, 2 * OUT), lambda i: (i, 0)),
            pl.BlockSpec((blk, 1), lambda i: (i, 0)),
            pl.BlockSpec((1, 2 * OUT), lambda i: (0, 0)),
            pl.BlockSpec((2 * OUT, FW), lambda i: (0, 0)),
        ],
        out_specs=pl.BlockSpec((blk, FW), lambda i: (i, 0)),
        out_shape=jax.ShapeDtypeStruct((N, FW), jnp.float32),
    )(p0, p1, hn1, dinv, b1_2, gcn2_w_pad)


def _t4_body(p0_ref, p1_ref, hn2_ref, dinv_ref, b2_ref, o_ref):
    s = p0_ref[...] + p1_ref[...] + hn2_ref[...]
    s = dinv_ref[...] * s[:, :OUT] + b2_ref[...]
    m = jnp.max(s, axis=1, keepdims=True)
    e = jnp.exp(s - m)
    o_ref[...] = e / jnp.sum(e, axis=1, keepdims=True)


def _finalize(p0, p1, hn2, dinv, b2_2):
    blk = 2000
    return pl.pallas_call(
        _t4_body,
        grid=(N // blk,),
        in_specs=[
            pl.BlockSpec((blk, FW), lambda i: (i, 0)),
            pl.BlockSpec((blk, FW), lambda i: (i, 0)),
            pl.BlockSpec((blk, FW), lambda i: (i, 0)),
            pl.BlockSpec((blk, 1), lambda i: (i, 0)),
            pl.BlockSpec((1, OUT), lambda i: (0, 0)),
        ],
        out_specs=pl.BlockSpec((blk, OUT), lambda i: (i, 0)),
        out_shape=jax.ShapeDtypeStruct((N, OUT), jnp.float32),
    )(p0, p1, hn2, dinv, b2_2)


def kernel(x, deg, edge_index, walks, w_in, wq, wk, wv, wo, w1, w2, w_se,
           gcn1_w, gcn1_b, gcn2_w, gcn2_b):
    f32 = jnp.float32
    wflat = jnp.concatenate([
        walks[:, :2, :].reshape(-1), walks[:, 2:, :].reshape(-1)
    ]).astype(jnp.int32)
    src = edge_index[0].astype(jnp.int32)
    dst = edge_index[1].astype(jnp.int32)

    ones_rows = jnp.ones((CHUNK, FW), f32)
    zeros_rows = jnp.zeros((NPAD // NS, FW), f32)

    w_in_pad = jnp.pad(w_in, ((0, 0), (0, FW - HID)))
    xp = _project(x, w_in_pad)
    z = _gather_rows(xp, wflat)
    hist = _dst_histogram(dst, ones_rows, zeros_rows)

    h = _transformer(
        z, deg.reshape(N, 1), wq[0], wk[0], wv[0], wo[0],
        w1[0], w2[0], w_se.reshape(1, HID), gcn1_w)
    hn1, dinv = _apply_dinv(h, hist[0, :N], hist[1, :N])

    s1 = _edge_scatter(hn1, src, dst, zeros_rows)
    gcn2_w_pad = jnp.pad(gcn2_w, ((0, 0), (0, FW - OUT)))
    hn2 = _gcn_mid(s1[0, :N], s1[1, :N], hn1, dinv,
                   gcn1_b.reshape(1, 2 * OUT), gcn2_w_pad)

    s2 = _edge_scatter(hn2, src, dst, zeros_rows)
    return _finalize(s2[0, :N], s2[1, :N], hn2, dinv,
                     gcn2_b.reshape(1, OUT))


# speedup vs baseline: 2.4620x; 1.0156x over previous
import functools

import jax
import jax.numpy as jnp
from jax import lax
from jax.experimental import pallas as pl
from jax.experimental.pallas import tpu as pltpu
from jax.experimental.pallas import tpu_sc as plsc

N = 10000
IN_DIM = 128
HID = 64
OUT = 64
NUM_WALKS = 4
WALK_LEN = 8
HEADS = 4
DH = HID // HEADS
N_EDGES = 320000
TOK = NUM_WALKS * WALK_LEN
NTOK = N * TOK
FW = 128

NPAD = 10240
NC, NS = 2, 16
NW = NC * NS
CHUNK = 80


def _t1_body(x_ref, w_ref, o_ref):
    o_ref[...] = jnp.dot(x_ref[...], w_ref[...],
                         preferred_element_type=jnp.float32)


def _project(x, w_in_pad):
    blk = 2000
    return pl.pallas_call(
        _t1_body,
        grid=(N // blk,),
        in_specs=[
            pl.BlockSpec((blk, IN_DIM), lambda i: (i, 0)),
            pl.BlockSpec((IN_DIM, FW), lambda i: (0, 0)),
        ],
        out_specs=pl.BlockSpec((blk, FW), lambda i: (i, 0)),
        out_shape=jax.ShapeDtypeStruct((N, FW), jnp.float32),
    )(x, w_in_pad)


def _gather_rows(xp, wflat):
    per_w = NTOK // NW
    nchunks = per_w // CHUNK
    rows_per_tile = NPAD // NS

    mesh = plsc.VectorSubcoreMesh(core_axis_name="c", subcore_axis_name="s")

    @functools.partial(
        pl.kernel, mesh=mesh,
        out_type=jax.ShapeDtypeStruct((NTOK, FW), jnp.float32),
        scratch_types=[
            pltpu.VMEM((CHUNK,), jnp.int32),
            pltpu.VMEM((CHUNK,), jnp.int32),
            pltpu.VMEM((CHUNK, FW), jnp.float32),
            pltpu.VMEM((CHUNK, FW), jnp.float32),
            pltpu.SemaphoreType.DMA,
            pltpu.SemaphoreType.DMA,
        ],
    )
    def k(xp_hbm, idx_hbm, z_hbm, i0, i1, r0, r1, sem0, sem1):
        cid = lax.axis_index("c")
        sid = lax.axis_index("s")
        wid = cid * NS + sid
        base = pl.multiple_of(wid * per_w, 8)

        def off(c):
            return pl.multiple_of(base + c * CHUNK, 8)

        def load(c, iv):
            pltpu.sync_copy(idx_hbm.at[pl.ds(off(c), CHUNK)], iv)

        def consume(c, iv, rv, sem):
            pltpu.make_async_copy(xp_hbm.at[iv], rv, sem).wait()
            pltpu.sync_copy(rv, z_hbm.at[pl.ds(off(c), CHUNK)])

        load(0, i0)
        pltpu.async_copy(xp_hbm.at[i0], r0, sem0)

        def body(j, _):
            c0 = 2 * j
            load(c0 + 1, i1)
            pltpu.async_copy(xp_hbm.at[i1], r1, sem1)
            consume(c0, i0, r0, sem0)
            load(c0 + 2, i0)
            pltpu.async_copy(xp_hbm.at[i0], r0, sem0)
            consume(c0 + 1, i1, r1, sem1)
            return 0

        lax.fori_loop(0, (nchunks - 1) // 2, body, 0)
        consume(nchunks - 1, i0, r0, sem0)

    return k(xp, wflat)


def _dst_histogram(dst, ones_rows, zeros_rows):
    per_w = N_EDGES // NW
    nchunks = per_w // CHUNK
    rows_per_tile = NPAD // NS

    mesh = plsc.VectorSubcoreMesh(core_axis_name="c", subcore_axis_name="s")

    @functools.partial(
        pl.kernel, mesh=mesh,
        out_type=jax.ShapeDtypeStruct((NC, NPAD, FW), jnp.float32),
        scratch_types=[
            pltpu.VMEM((CHUNK,), jnp.int32),
            pltpu.VMEM((CHUNK, FW), jnp.float32),
            pltpu.VMEM_SHARED((NPAD, FW), jnp.float32),
        ],
    )
    def k(dst_hbm, ones_hbm, zeros_hbm, hist_hbm, didx_v, ones_v, acc_sh):
        cid = lax.axis_index("c")
        sid = lax.axis_index("s")
        wid = cid * NS + sid
        rbase = pl.multiple_of(sid * rows_per_tile, 8)
        pltpu.sync_copy(zeros_hbm.at[pl.ds(0, rows_per_tile)],
                        acc_sh.at[pl.ds(rbase, rows_per_tile)])
        pltpu.sync_copy(ones_hbm, ones_v)
        plsc.subcore_barrier()

        base = pl.multiple_of(wid * per_w, 8)

        def body(j, _):
            off = pl.multiple_of(base + j * CHUNK, 8)
            pltpu.sync_copy(dst_hbm.at[pl.ds(off, CHUNK)], didx_v)
            pltpu.sync_copy(ones_v, acc_sh.at[didx_v], add=True)
            return 0

        lax.fori_loop(0, nchunks, body, 0)
        plsc.subcore_barrier()
        pltpu.sync_copy(acc_sh.at[pl.ds(rbase, rows_per_tile)],
                        hist_hbm.at[cid, pl.ds(rbase, rows_per_tile)])

    return k(dst, ones_rows, zeros_rows)


def _edge_scatter(rows, src, dst, zeros_rows):
    per_w = N_EDGES // NW
    nchunks = per_w // CHUNK
    rows_per_tile = NPAD // NS

    mesh = plsc.VectorSubcoreMesh(core_axis_name="c", subcore_axis_name="s")

    @functools.partial(
        pl.kernel, mesh=mesh,
        out_type=jax.ShapeDtypeStruct((NC, NPAD, FW), jnp.float32),
        scratch_types=[
            pltpu.VMEM((CHUNK,), jnp.int32),
            pltpu.VMEM((CHUNK,), jnp.int32),
            pltpu.VMEM((CHUNK,), jnp.int32),
            pltpu.VMEM((CHUNK,), jnp.int32),
            pltpu.VMEM((CHUNK, FW), jnp.float32),
            pltpu.VMEM((CHUNK, FW), jnp.float32),
            pltpu.VMEM_SHARED((NPAD, FW), jnp.float32),
            pltpu.SemaphoreType.DMA,
            pltpu.SemaphoreType.DMA,
        ],
    )
    def k(rows_hbm, src_hbm, dst_hbm, zeros_hbm, out_hbm,
          s0, s1, d0, d1, r0, r1, acc_sh, sem0, sem1):
        cid = lax.axis_index("c")
        sid = lax.axis_index("s")
        wid = cid * NS + sid
        rbase = pl.multiple_of(sid * rows_per_tile, 8)
        pltpu.sync_copy(zeros_hbm.at[pl.ds(0, rows_per_tile)],
                        acc_sh.at[pl.ds(rbase, rows_per_tile)])
        plsc.subcore_barrier()

        base = pl.multiple_of(wid * per_w, 8)

        def off(c):
            return pl.multiple_of(base + c * CHUNK, 8)

        def load(c, sv, dv):
            pltpu.sync_copy(src_hbm.at[pl.ds(off(c), CHUNK)], sv)
            pltpu.sync_copy(dst_hbm.at[pl.ds(off(c), CHUNK)], dv)

        def consume(sv, dv, rv, sem):
            pltpu.make_async_copy(rows_hbm.at[sv], rv, sem).wait()
            pltpu.sync_copy(rv, acc_sh.at[dv], add=True)

        load(0, s0, d0)
        pltpu.async_copy(rows_hbm.at[s0], r0, sem0)

        def body(j, _):
            c0 = 2 * j
            load(c0 + 1, s1, d1)
            pltpu.async_copy(rows_hbm.at[s1], r1, sem1)
            consume(s0, d0, r0, sem0)
            load(c0 + 2, s0, d0)
            pltpu.async_copy(rows_hbm.at[s0], r0, sem0)
            consume(s1, d1, r1, sem1)
            return 0

        lax.fori_loop(0, (nchunks - 1) // 2, body, 0)
        consume(s0, d0, r0, sem0)
        plsc.subcore_barrier()
        pltpu.sync_copy(acc_sh.at[pl.ds(rbase, rows_per_tile)],
                        out_hbm.at[cid, pl.ds(rbase, rows_per_tile)])

    return k(rows, src, dst, zeros_rows)


BN = 200
BR = BN * TOK // 2
NROW = NTOK // 2
NBLK = N // BN


def _group_roll(arr, o, pos):
    t = arr.shape[0]
    a = jnp.concatenate([arr[o:], arr[:o]], axis=0)
    r2 = t + o - WALK_LEN
    b = jnp.concatenate([arr[r2:], arr[:r2]], axis=0)
    return jnp.where(pos < WALK_LEN - o, a, b)


def _t2_body(za_ref, zb_ref, deg_ref, wq_ref, wk_ref, wv_ref,
             wo_ref, w1_ref, w2_ref, e2_ref, e2t_ref, wse_ref, g1_ref,
             h_ref):
    f32 = jnp.float32
    z = jnp.concatenate([za_ref[...][:, :HID], zb_ref[...][:, :HID]], axis=1)
    q = jnp.dot(z, wq_ref[...], preferred_element_type=f32) * (1.0 / 4.0)
    kk = jnp.dot(z, wk_ref[...], preferred_element_type=f32)
    v = jnp.dot(z, wv_ref[...], preferred_element_type=f32)

    E2 = e2_ref[...]
    E2T = e2t_ref[...]
    pos = lax.broadcasted_iota(jnp.int32, (BR, FW), 0) % WALK_LEN

    ks = [kk] + [_group_roll(kk, o, pos) for o in range(1, WALK_LEN)]
    logits = [jnp.dot(q * ko, E2, preferred_element_type=f32) for ko in ks]
    lcat = jnp.concatenate(logits, axis=1)
    e = jnp.exp(lcat)
    s8 = (lax.broadcasted_iota(jnp.int32, (8 * WALK_LEN, 8), 0) % 8 ==
          lax.broadcasted_iota(jnp.int32, (8 * WALK_LEN, 8), 1)).astype(f32)
    den8 = jnp.dot(e, s8, preferred_element_type=f32)
    inv_full = jnp.dot(1.0 / den8, E2T,
                       preferred_element_type=f32)
    o_acc = None
    for o in range(WALK_LEN):
        e_full = jnp.dot(e[:, 8 * o:8 * o + 8], E2T,
                         preferred_element_type=f32)
        vo = v if o == 0 else _group_roll(v, o, pos)
        contrib = e_full * vo
        o_acc = contrib if o_acc is None else o_acc + contrib
    o_acc = o_acc * inv_full
    z = z + jnp.dot(o_acc, wo_ref[...], preferred_element_type=f32)
    h1 = jnp.maximum(jnp.dot(z, w1_ref[...], preferred_element_type=f32), 0.0)
    z = z + jnp.dot(h1, w2_ref[...], preferred_element_type=f32)

    ri = lax.broadcasted_iota(jnp.int32, (BN, BR), 0)
    ci = lax.broadcasted_iota(jnp.int32, (BN, BR), 1) // (TOK // 2)
    P = jnp.where(ri == ci, 1.0 / TOK, 0.0).astype(f32)
    p128 = jnp.dot(P, z, preferred_element_type=f32)
    pooled = p128[:, :HID] + p128[:, HID:]

    deg = deg_ref[...]
    gf = 1.0 + jnp.log1p(jnp.maximum(deg, 0.0)) * wse_ref[...]
    gt = jnp.maximum(pooled * gf, 0.0)

    h_ref[...] = jnp.dot(gt, g1_ref[...], preferred_element_type=f32)


def _bd(w):
    a, b = w.shape
    z = jnp.zeros((a, b), w.dtype)
    return jnp.concatenate([
        jnp.concatenate([w, z], axis=1),
        jnp.concatenate([z, w], axis=1),
    ], axis=0)


def _transformer(z, deg2, wq, wk, wv, wo, w1, w2, wse2, gcn1_w):
    f32 = jnp.float32
    bd_wq, bd_wk, bd_wv, bd_wo = _bd(wq), _bd(wk), _bd(wv), _bd(wo)
    bd_w1, bd_w2 = _bd(w1), _bd(w2)
    li = jnp.arange(FW)
    hj = jnp.arange(2 * HEADS)
    e2 = (li[:, None] // DH == hj[None, :]).astype(f32)
    e2t = e2.T
    wcon = pl.BlockSpec((FW, FW), lambda i: (0, 0))
    return pl.pallas_call(
        _t2_body,
        grid=(NBLK,),
        in_specs=[
            pl.BlockSpec((BR, FW), lambda i: (i, 0)),
            pl.BlockSpec((BR, FW), lambda i: (i + NBLK, 0)),
            pl.BlockSpec((BN, 1), lambda i: (i, 0)),
            wcon, wcon, wcon, wcon,
            pl.BlockSpec((FW, 2 * FW), lambda i: (0, 0)),
            pl.BlockSpec((2 * FW, FW), lambda i: (0, 0)),
            pl.BlockSpec((FW, 2 * HEADS), lambda i: (0, 0)),
            pl.BlockSpec((2 * HEADS, FW), lambda i: (0, 0)),
            pl.BlockSpec((1, HID), lambda i: (0, 0)),
            pl.BlockSpec((HID, 2 * OUT), lambda i: (0, 0)),
        ],
        out_specs=pl.BlockSpec((BN, 2 * OUT), lambda i: (i, 0)),
        out_shape=jax.ShapeDtypeStruct((N, 2 * OUT), jnp.float32),
    )(z, z, deg2, bd_wq, bd_wk, bd_wv, bd_wo, bd_w1, bd_w2,
      e2, e2t, wse2, gcn1_w)


def _t2b_body(h_ref, h0_ref, h1_ref, hn1_ref, dinv_ref):
    degc = h0_ref[...][:, 0:1] + h1_ref[...][:, 0:1] + 1.0
    dinv = lax.rsqrt(degc)
    hn1_ref[...] = h_ref[...] * dinv
    dinv_ref[...] = dinv


def _apply_dinv(h, hist0, hist1):
    blk = 2000
    return pl.pallas_call(
        _t2b_body,
        grid=(N // blk,),
        in_specs=[
            pl.BlockSpec((blk, FW), lambda i: (i, 0)),
            pl.BlockSpec((blk, FW), lambda i: (i, 0)),
            pl.BlockSpec((blk, FW), lambda i: (i, 0)),
        ],
        out_specs=[
            pl.BlockSpec((blk, FW), lambda i: (i, 0)),
            pl.BlockSpec((blk, 1), lambda i: (i, 0)),
        ],
        out_shape=[
            jax.ShapeDtypeStruct((N, FW), jnp.float32),
            jax.ShapeDtypeStruct((N, 1), jnp.float32),
        ],
    )(h, hist0, hist1)


def _t3_body(p0_ref, p1_ref, hn1_ref, dinv_ref, b1_ref, g2_ref, hn2_ref):
    dinv = dinv_ref[...]
    s = p0_ref[...] + p1_ref[...] + hn1_ref[...]
    h1 = jnp.maximum(dinv * s + b1_ref[...], 0.0)
    hn2_ref[...] = dinv * jnp.dot(h1, g2_ref[...],
                                  preferred_element_type=jnp.float32)


def _gcn_mid(p0, p1, hn1, dinv, b1_2, gcn2_w_pad):
    blk = 2000
    return pl.pallas_call(
        _t3_body,
        grid=(N // blk,),
        in_specs=[
            pl.BlockSpec((blk, FW), lambda i: (i, 0)),
            pl.BlockSpec((blk, FW), lambda i: (i, 0)),
            pl.BlockSpec((blk, 2 * OUT), lambda i: (i, 0)),
            pl.BlockSpec((blk, 1), lambda i: (i, 0)),
            pl.BlockSpec((1, 2 * OUT), lambda i: (0, 0)),
            pl.BlockSpec((2 * OUT, FW), lambda i: (0, 0)),
        ],
        out_specs=pl.BlockSpec((blk, FW), lambda i: (i, 0)),
        out_shape=jax.ShapeDtypeStruct((N, FW), jnp.float32),
    )(p0, p1, hn1, dinv, b1_2, gcn2_w_pad)


def _t4_body(p0_ref, p1_ref, hn2_ref, dinv_ref, b2_ref, o_ref):
    s = p0_ref[...] + p1_ref[...] + hn2_ref[...]
    s = dinv_ref[...] * s[:, :OUT] + b2_ref[...]
    m = jnp.max(s, axis=1, keepdims=True)
    e = jnp.exp(s - m)
    o_ref[...] = e / jnp.sum(e, axis=1, keepdims=True)


def _finalize(p0, p1, hn2, dinv, b2_2):
    blk = 2000
    return pl.pallas_call(
        _t4_body,
        grid=(N // blk,),
        in_specs=[
            pl.BlockSpec((blk, FW), lambda i: (i, 0)),
            pl.BlockSpec((blk, FW), lambda i: (i, 0)),
            pl.BlockSpec((blk, FW), lambda i: (i, 0)),
            pl.BlockSpec((blk, 1), lambda i: (i, 0)),
            pl.BlockSpec((1, OUT), lambda i: (0, 0)),
        ],
        out_specs=pl.BlockSpec((blk, OUT), lambda i: (i, 0)),
        out_shape=jax.ShapeDtypeStruct((N, OUT), jnp.float32),
    )(p0, p1, hn2, dinv, b2_2)


def kernel(x, deg, edge_index, walks, w_in, wq, wk, wv, wo, w1, w2, w_se,
           gcn1_w, gcn1_b, gcn2_w, gcn2_b):
    f32 = jnp.float32
    wflat = jnp.concatenate([
        walks[:, :2, :].reshape(-1), walks[:, 2:, :].reshape(-1)
    ]).astype(jnp.int32)
    src = edge_index[0].astype(jnp.int32)
    dst = edge_index[1].astype(jnp.int32)

    ones_rows = jnp.ones((CHUNK, FW), f32)
    zeros_rows = jnp.zeros((NPAD // NS, FW), f32)

    w_in_pad = jnp.pad(w_in, ((0, 0), (0, FW - HID)))
    xp = _project(x, w_in_pad)
    z = _gather_rows(xp, wflat)
    hist = _dst_histogram(dst, ones_rows, zeros_rows)

    h = _transformer(
        z, deg.reshape(N, 1), wq[0], wk[0], wv[0], wo[0],
        w1[0], w2[0], w_se.reshape(1, HID), gcn1_w)
    hn1, dinv = _apply_dinv(h, hist[0, :N], hist[1, :N])

    s1 = _edge_scatter(hn1, src, dst, zeros_rows)
    gcn2_w_pad = jnp.pad(gcn2_w, ((0, 0), (0, FW - OUT)))
    hn2 = _gcn_mid(s1[0, :N], s1[1, :N], hn1, dinv,
                   gcn1_b.reshape(1, 2 * OUT), gcn2_w_pad)

    s2 = _edge_scatter(hn2, src, dst, zeros_rows)
    return _finalize(s2[0, :N], s2[1, :N], hn2, dinv,
                     gcn2_b.reshape(1, OUT))
